# baseline (speedup 1.0000x reference)
import functools

import jax
import jax.numpy as jnp
from jax.experimental import pallas as pl
from jax.experimental.pallas import tpu as pltpu
from jax.experimental.pallas import tpu_sc as plsc

B = 4
N = 2048
KNN = 30
F = 64
T_SEL = 256
T_EDGE = 256
T_MLP = 512
CLEAR = 1e30
BIGCOL = 1e9
NW = 32
CH = 120


def _uv_body(x_ref, w1u_ref, w1v_ref, b1_ref, u_ref, v_ref):
    x = x_ref[0]
    u_ref[0] = jnp.dot(x, w1u_ref[...], preferred_element_type=jnp.float32) + b1_ref[...]
    v_ref[0] = jnp.dot(x, w1v_ref[...], preferred_element_type=jnp.float32)


def _uv_call(x, w1u, w1v, b1):
    dp = x.shape[-1]
    return pl.pallas_call(
        _uv_body,
        grid=(B,),
        in_specs=[
            pl.BlockSpec((1, N, dp), lambda b: (b, 0, 0)),
            pl.BlockSpec((dp, F), lambda b: (0, 0)),
            pl.BlockSpec((dp, F), lambda b: (0, 0)),
            pl.BlockSpec((1, F), lambda b: (0, 0)),
        ],
        out_specs=[
            pl.BlockSpec((1, N, F), lambda b: (b, 0, 0)),
            pl.BlockSpec((1, N, F), lambda b: (b, 0, 0)),
        ],
        out_shape=[
            jax.ShapeDtypeStruct((B, N, F), jnp.float32),
            jax.ShapeDtypeStruct((B, N, F), jnp.float32),
        ],
    )(x, w1u, w1v, b1)


def _select_body(xt_ref, xT_ref, v_ref, u_ref, idx_ref, stats_ref):
    b = pl.program_id(0)
    tile = pl.program_id(1)
    xt = xt_ref[0]
    xT = xT_ref[0]

    r = jnp.dot(xt, xT, preferred_element_type=jnp.float32)
    d2t = jnp.sum(xt * xt, axis=1, keepdims=True)
    d2row = jnp.sum(xT * xT, axis=0, keepdims=True)
    dist = (d2t + d2row) - 2.0 * r

    col_i = jax.lax.broadcasted_iota(jnp.int32, (T_SEL, N), 1)
    row_i = jax.lax.broadcasted_iota(jnp.int32, (T_SEL, N), 0) + tile * T_SEL
    dist = dist + jnp.where(col_i == row_i, 1e10, 0.0)

    lane_i = jax.lax.broadcasted_iota(jnp.int32, (T_SEL, 32), 1)

    def step(t, carry):
        d, sel = carry
        j = jnp.argmin(d, axis=1).reshape(T_SEL, 1)
        sel = jnp.where(lane_i == t, j, sel)
        d = jnp.where(col_i == j, CLEAR, d)
        return d, sel

    sel0 = jnp.zeros((T_SEL, 32), jnp.int32)
    dist, sel = jax.lax.fori_loop(0, KNN, step, (dist, sel0))

    idx_ref[0] = sel[:, :KNN] + b * N

    chosen = (dist >= 1e29).astype(jnp.float32)
    v = v_ref[0]
    s = jnp.dot(chosen, v, preferred_element_type=jnp.float32)
    cm = jnp.sum(chosen, axis=0, keepdims=True)
    q1 = jnp.dot(cm, v * v, preferred_element_type=jnp.float32)
    u = u_ref[0]
    r0 = jnp.sum(s, axis=0, keepdims=True)
    r2 = jnp.sum(u, axis=0, keepdims=True)
    r3 = jnp.sum(u * u, axis=0, keepdims=True)
    r4 = jnp.sum(u * s, axis=0, keepdims=True)
    z3 = jnp.zeros((3, F), jnp.float32)
    stats = jnp.concatenate([r0, q1, r2, r3, r4, z3], axis=0)

    @pl.when(jnp.logical_and(b == 0, tile == 0))
    def _():
        stats_ref[...] = jnp.zeros((8, F), jnp.float32)

    stats_ref[...] += stats


def _select_call(x, xT, v, u):
    dp = x.shape[-1]
    nt = N // T_SEL
    return pl.pallas_call(
        _select_body,
        grid=(B, nt),
        in_specs=[
            pl.BlockSpec((1, T_SEL, dp), lambda b, t: (b, t, 0)),
            pl.BlockSpec((1, dp, N), lambda b, t: (b, 0, 0)),
            pl.BlockSpec((1, N, F), lambda b, t: (b, 0, 0)),
            pl.BlockSpec((1, T_SEL, F), lambda b, t: (b, t, 0)),
        ],
        out_specs=[
            pl.BlockSpec((1, T_SEL, KNN), lambda b, t: (b, t, 0)),
            pl.BlockSpec((8, F), lambda b, t: (0, 0)),
        ],
        out_shape=[
            jax.ShapeDtypeStruct((B, N, KNN), jnp.int32),
            jax.ShapeDtypeStruct((8, F), jnp.float32),
        ],
    )(x, xT, v, u)


def _make_sc_gather(n_idx, d):
    per_w = n_idx // NW
    nch = per_w // CH
    mesh = plsc.VectorSubcoreMesh(core_axis_name="c", subcore_axis_name="s")

    @functools.partial(
        pl.kernel,
        mesh=mesh,
        out_type=jax.ShapeDtypeStruct((n_idx, d), jnp.float32),
        scratch_types=[
            pltpu.VMEM((nch, CH), jnp.int32),
            pltpu.VMEM((CH, d), jnp.float32),
            pltpu.VMEM((CH, d), jnp.float32),
            pltpu.SemaphoreType.DMA,
        ],
        compiler_params=pltpu.CompilerParams(use_tc_tiling_on_sc=False),
    )
    def gk(table_hbm, idx_hbm, out_hbm, idx_v, buf0, buf1, sem):
        wid = jax.lax.axis_index("s") * 2 + jax.lax.axis_index("c")
        rbase = pl.multiple_of(wid * per_w, 8)
        cbase = pl.multiple_of(wid * nch, 8)
        pltpu.sync_copy(idx_hbm.at[pl.ds(cbase, nch)], idx_v)
        bufs = (buf0, buf1)
        pltpu.async_copy(table_hbm.at[idx_v.at[0]], buf0, sem)

        def outer(oc, _):
            c0 = oc * 2
            for bb in range(2):
                c = c0 + bb
                cur = bufs[bb]
                nxt = bufs[(bb + 1) % 2]

                @pl.when(c + 1 < nch)
                def _():
                    pltpu.async_copy(table_hbm.at[idx_v.at[c + 1]], nxt, sem)

                pltpu.make_async_copy(table_hbm.at[idx_v.at[c]], cur, sem).wait()
                roff = pl.multiple_of(rbase + c * CH, 8)
                pltpu.sync_copy(cur, out_hbm.at[pl.ds(roff, CH)])
            return 0

        jax.lax.fori_loop(0, nch // 2, outer, 0)

    return gk


def _gather_rows(table, idx2d):
    n_idx = idx2d.shape[0] * idx2d.shape[1]
    return _make_sc_gather(n_idx, table.shape[-1])(table, idx2d)


def _edge_body(vg_ref, u_ref, a_ref, c_ref, w2_ref, b2_ref, out_ref):
    u = u_ref[0]
    a = a_ref[...]
    c = c_ref[...]
    w2 = w2_ref[...]
    acc = jnp.full((T_EDGE, F), -1e30, jnp.float32)
    for t in range(KNN):
        vt = vg_ref[0, :, t, :]
        z = jax.nn.relu((u + vt) * a + c)
        y = jnp.dot(z, w2, preferred_element_type=jnp.float32)
        acc = jnp.maximum(acc, y)
    out_ref[0] = acc + b2_ref[...]


def _edge_call(vg4, u, a, c, w2, b2):
    nt = N // T_EDGE
    return pl.pallas_call(
        _edge_body,
        grid=(B, nt),
        in_specs=[
            pl.BlockSpec((1, T_EDGE, KNN, F), lambda b, t: (b, t, 0, 0)),
            pl.BlockSpec((1, T_EDGE, F), lambda b, t: (b, t, 0)),
            pl.BlockSpec((1, F), lambda b, t: (0, 0)),
            pl.BlockSpec((1, F), lambda b, t: (0, 0)),
            pl.BlockSpec((F, F), lambda b, t: (0, 0)),
            pl.BlockSpec((1, F), lambda b, t: (0, 0)),
        ],
        out_specs=pl.BlockSpec((1, T_EDGE, F), lambda b, t: (b, t, 0)),
        out_shape=jax.ShapeDtypeStruct((B, N, F), jnp.float32),
    )(vg4, u, a, c, w2, b2)


def _mlp_body(x_ref, w1_ref, b1_ref, w2_ref, b2_ref, w3_ref, b3_ref,
              w4_ref, b4_ref, out_ref):
    h = jax.nn.relu(jnp.dot(x_ref[...], w1_ref[...], preferred_element_type=jnp.float32) + b1_ref[...])
    h = jax.nn.relu(jnp.dot(h, w2_ref[...], preferred_element_type=jnp.float32) + b2_ref[...])
    h = jax.nn.relu(jnp.dot(h, w3_ref[...], preferred_element_type=jnp.float32) + b3_ref[...])
    o = jnp.dot(h, w4_ref[...], preferred_element_type=jnp.float32) + b4_ref[...]
    m = jnp.max(o, axis=1, keepdims=True)
    sh = o - m
    out_ref[...] = sh - jnp.log(jnp.sum(jnp.exp(sh), axis=1, keepdims=True))


def _mlp_call(x, w1, b1, w2, b2, w3, b3, w4, b4):
    rows = x.shape[0]
    nt = rows // T_MLP
    ncls = w4.shape[-1]
    return pl.pallas_call(
        _mlp_body,
        grid=(nt,),
        in_specs=[
            pl.BlockSpec((T_MLP, x.shape[1]), lambda i: (i, 0)),
            pl.BlockSpec(w1.shape, lambda i: (0, 0)),
            pl.BlockSpec((1, w1.shape[1]), lambda i: (0, 0)),
            pl.BlockSpec(w2.shape, lambda i: (0, 0)),
            pl.BlockSpec((1, w2.shape[1]), lambda i: (0, 0)),
            pl.BlockSpec(w3.shape, lambda i: (0, 0)),
            pl.BlockSpec((1, w3.shape[1]), lambda i: (0, 0)),
            pl.BlockSpec(w4.shape, lambda i: (0, 0)),
            pl.BlockSpec((1, ncls), lambda i: (0, 0)),
        ],
        out_specs=pl.BlockSpec((T_MLP, ncls), lambda i: (i, 0)),
        out_shape=jax.ShapeDtypeStruct((rows, ncls), jnp.float32),
    )(x, w1, b1, w2, b2, w3, b3, w4, b4)


def _edge_conv_layer(x, W1, b1, g, be, W2, b2):
    din = x.shape[-1]
    w1a, w1b = W1[:din], W1[din:]
    w1u = w1a - w1b
    dp = din
    if din % 8 != 0:
        pad = 8 - din % 8
        dp = din + pad
        x = jnp.pad(x, ((0, 0), (0, 0), (0, pad)))
        w1u = jnp.pad(w1u, ((0, pad), (0, 0)))
        w1b = jnp.pad(w1b, ((0, pad), (0, 0)))

    u, v = _uv_call(x, w1u, w1b, b1.reshape(1, F))

    xT = jnp.swapaxes(x, 1, 2)
    idx, stats = _select_call(x, xT, v, u)

    vg = _gather_rows(v.reshape(B * N, F), idx.reshape(-1, CH))
    vg4 = vg.reshape(B, N, KNN, F)

    n_edges = B * N * KNN
    ss, sq, su, su2, sus = stats[0], stats[1], stats[2], stats[3], stats[4]
    mu = (KNN * su + ss) / n_edges
    msq = (KNN * su2 + 2.0 * sus + sq) / n_edges
    var = msq - mu * mu
    a = g / jnp.sqrt(var + 1e-5)
    c = be - mu * a

    return _edge_call(vg4, u, a.reshape(1, F), c.reshape(1, F), W2,
                      b2.reshape(1, F))


def kernel(data, c1_W1, c1_b1, c1_g, c1_be, c1_W2, c1_b2, c2_W1, c2_b1, c2_g,
           c2_be, c2_W2, c2_b2, c3_W1, c3_b1, c3_g, c3_be, c3_W2, c3_b2, m_W1,
           m_b1, m_W2, m_b2, m_W3, m_b3, m_W4, m_b4):
    x0 = data
    x1 = _edge_conv_layer(x0, c1_W1, c1_b1, c1_g, c1_be, c1_W2, c1_b2)
    x2 = _edge_conv_layer(x1, c2_W1, c2_b1, c2_g, c2_be, c2_W2, c2_b2)
    x3 = _edge_conv_layer(x2, c3_W1, c3_b1, c3_g, c3_be, c3_W2, c3_b2)
    h = jnp.concatenate([x1, x2, x3], axis=-1).reshape(B * N, 3 * F)
    return _mlp_call(h, m_W1, m_b1.reshape(1, -1), m_W2, m_b2.reshape(1, -1),
                     m_W3, m_b3.reshape(1, -1), m_W4, m_b4.reshape(1, -1))


# speedup vs baseline: 1.5301x; 1.5301x over previous
import functools

import jax
import jax.numpy as jnp
from jax.experimental import pallas as pl
from jax.experimental.pallas import tpu as pltpu
from jax.experimental.pallas import tpu_sc as plsc

B = 4
N = 2048
KNN = 30
F = 64
T_SEL = 256
T_EDGE = 256
T_MLP = 512
CLEAR = 1e30
BIGCOL = 1e9
NW = 32
CH = 120


def _uv_body(x_ref, xT_ref, w1u_ref, w1v_ref, w1uT_ref, w1vT_ref, b1_ref,
             b1c_ref, u_ref, v_ref, ut_ref, vt_ref):
    x = x_ref[0]
    xT = xT_ref[0]
    u_ref[0] = jnp.dot(x, w1u_ref[...], preferred_element_type=jnp.float32) + b1_ref[...]
    v_ref[0] = jnp.dot(x, w1v_ref[...], preferred_element_type=jnp.float32)
    ut_ref[0] = jnp.dot(w1uT_ref[...], xT, preferred_element_type=jnp.float32) + b1c_ref[...]
    vt_ref[0] = jnp.dot(w1vT_ref[...], xT, preferred_element_type=jnp.float32)


def _uv_call(x, xT, w1u, w1v, b1):
    dp = x.shape[-1]
    return pl.pallas_call(
        _uv_body,
        grid=(B,),
        in_specs=[
            pl.BlockSpec((1, N, dp), lambda b: (b, 0, 0)),
            pl.BlockSpec((1, dp, N), lambda b: (b, 0, 0)),
            pl.BlockSpec((dp, F), lambda b: (0, 0)),
            pl.BlockSpec((dp, F), lambda b: (0, 0)),
            pl.BlockSpec((F, dp), lambda b: (0, 0)),
            pl.BlockSpec((F, dp), lambda b: (0, 0)),
            pl.BlockSpec((1, F), lambda b: (0, 0)),
            pl.BlockSpec((F, 1), lambda b: (0, 0)),
        ],
        out_specs=[
            pl.BlockSpec((1, N, F), lambda b: (b, 0, 0)),
            pl.BlockSpec((1, N, F), lambda b: (b, 0, 0)),
            pl.BlockSpec((1, F, N), lambda b: (b, 0, 0)),
            pl.BlockSpec((1, F, N), lambda b: (b, 0, 0)),
        ],
        out_shape=[
            jax.ShapeDtypeStruct((B, N, F), jnp.float32),
            jax.ShapeDtypeStruct((B, N, F), jnp.float32),
            jax.ShapeDtypeStruct((B, F, N), jnp.float32),
            jax.ShapeDtypeStruct((B, F, N), jnp.float32),
        ],
    )(x, xT, w1u, w1v, w1u.T, w1v.T, b1, b1.reshape(F, 1))


NCHK = 16
CW = N // NCHK
MPER = 12
RANKSTEP = 1e27
CHOSEN_MIN = 1e29
BIGP = 2e30


def _select_body(x_ref, xT_ref, vt_ref, ut_ref, idx_ref, stats_ref):
    b = pl.program_id(0)
    tile = pl.program_id(1)
    xTt = xT_ref[0]
    d2col = jnp.sum(xTt * xTt, axis=0, keepdims=True)
    boff_f = jax.lax.convert_element_type(b * N, jnp.float32)

    rowi = jax.lax.broadcasted_iota(jnp.int32, (CW, T_SEL), 0)
    coli = jax.lax.broadcasted_iota(jnp.int32, (CW, T_SEL), 1)
    rowf = rowi.astype(jnp.float32)
    subi16 = jax.lax.broadcasted_iota(jnp.int32, (16, T_SEL), 0)
    subi32 = jax.lax.broadcasted_iota(jnp.int32, (32, T_SEL), 0)

    def chunk_dist(c):
        xc = x_ref[0, c * CW:(c + 1) * CW, :]
        r = jnp.dot(xc, xTt, preferred_element_type=jnp.float32)
        d2c = jnp.sum(xc * xc, axis=1, keepdims=True)
        d = (d2c + d2col) - 2.0 * r
        return d + jnp.where(rowi + c * CW == coli + tile * T_SEL, 1e10, 0.0)

    d_enc, pvs, pis = [], [], []
    for c in range(NCHK):
        off_f = jnp.float32(c * CW) + boff_f

        def step(s, carry, off_f=off_f):
            d, pv, pi = carry
            mv = jnp.min(d, axis=0, keepdims=True)
            cand = jnp.where(d <= mv, rowf, BIGCOL)
            jr = jnp.min(cand, axis=0, keepdims=True)
            pv = jnp.where(subi16 == s, mv, pv)
            pi = jnp.where(subi16 == s, jr + off_f, pi)
            mark = CLEAR + jax.lax.convert_element_type(s, jnp.float32) * RANKSTEP
            d = jnp.where(rowf == jr, mark, d)
            return d, pv, pi

        pv0 = jnp.full((16, T_SEL), BIGP, jnp.float32)
        pi0 = jnp.full((16, T_SEL), -1.0, jnp.float32)
        d, pv, pi = jax.lax.fori_loop(0, MPER, step, (chunk_dist(c), pv0, pi0))
        d_enc.append(d)
        pvs.append(pv)
        pis.append(pi)

    P0 = jnp.concatenate(pvs, axis=0)
    PI = jnp.concatenate(pis, axis=0)

    def pstep(t, carry):
        P, pm, idxacc = carry
        wv = jnp.min(P, axis=0, keepdims=True)
        wi = jnp.min(jnp.where(P <= wv, PI, BIGCOL), axis=0, keepdims=True)
        hit = PI == wi
        pm = pm + hit.astype(jnp.float32)
        P = jnp.where(hit, BIGP, P)
        idxacc = jnp.where(subi32 == t, wi, idxacc)
        return P, pm, idxacc

    _, pm, idxacc = jax.lax.fori_loop(
        0, KNN, pstep,
        (P0, jnp.zeros((16 * NCHK, T_SEL), jnp.float32),
         jnp.zeros((32, T_SEL), jnp.float32)))

    ncs = [jnp.sum(pm[c * 16:(c + 1) * 16, :], axis=0, keepdims=True)
           for c in range(NCHK)]
    fb = jnp.max(jnp.concatenate(ncs, axis=0)) >= float(MPER)

    def normal_fn():
        parts = []
        for c in range(NCHK):
            thr = CLEAR + (ncs[c] - 0.5) * RANKSTEP
            ch = jnp.logical_and(d_enc[c] >= CHOSEN_MIN, d_enc[c] < thr)
            parts.append(ch.astype(jnp.float32))
        return (idxacc, tuple(parts))

    def fb_fn():
        def fstep(t, carry):
            idxa = carry[0]
            dd = carry[1:]
            mvs = [jnp.min(dc, axis=0, keepdims=True) for dc in dd]
            g = functools.reduce(jnp.minimum, mvs)
            cands = [jnp.min(jnp.where(dd[c] <= g, rowf + jnp.float32(c * CW),
                                       BIGCOL), axis=0, keepdims=True)
                     for c in range(NCHK)]
            wi = functools.reduce(jnp.minimum, cands)
            idxa = jnp.where(subi32 == t, wi + boff_f, idxa)
            dd = tuple(jnp.where(rowf + jnp.float32(c * CW) == wi, CLEAR, dd[c])
                       for c in range(NCHK))
            return (idxa,) + dd

        init = (jnp.zeros((32, T_SEL), jnp.float32),) + tuple(
            chunk_dist(c) for c in range(NCHK))
        out = jax.lax.fori_loop(0, KNN, fstep, init)
        parts = tuple((dc >= CHOSEN_MIN).astype(jnp.float32) for dc in out[1:])
        return (out[0], parts)

    idxf, chosen_parts = jax.lax.cond(fb, fb_fn, normal_fn)
    idx_ref[0] = idxf[:KNN].astype(jnp.int32)

    vt = vt_ref[0]
    st = jnp.zeros((F, T_SEL), jnp.float32)
    qt = jnp.zeros((F, T_SEL), jnp.float32)
    for c in range(NCHK):
        vc = vt[:, c * CW:(c + 1) * CW]
        st = st + jnp.dot(vc, chosen_parts[c], preferred_element_type=jnp.float32)
        qt = qt + jnp.dot(vc * vc, chosen_parts[c], preferred_element_type=jnp.float32)
    ut = ut_ref[0]
    c0 = jnp.sum(st, axis=1, keepdims=True)
    c1 = jnp.sum(qt, axis=1, keepdims=True)
    c2 = jnp.sum(ut, axis=1, keepdims=True)
    c3 = jnp.sum(ut * ut, axis=1, keepdims=True)
    c4 = jnp.sum(ut * st, axis=1, keepdims=True)
    stats = jnp.concatenate(
        [c0, c1, c2, c3, c4, jnp.zeros((F, 3), jnp.float32)], axis=1)

    @pl.when(jnp.logical_and(b == 0, tile == 0))
    def _():
        stats_ref[...] = jnp.zeros((F, 8), jnp.float32)

    stats_ref[...] += stats


def _select_call(x, xT, vt, ut):
    dp = x.shape[-1]
    nt = N // T_SEL
    return pl.pallas_call(
        _select_body,
        grid=(B, nt),
        in_specs=[
            pl.BlockSpec((1, N, dp), lambda b, t: (b, 0, 0)),
            pl.BlockSpec((1, dp, T_SEL), lambda b, t: (b, 0, t)),
            pl.BlockSpec((1, F, N), lambda b, t: (b, 0, 0)),
            pl.BlockSpec((1, F, T_SEL), lambda b, t: (b, 0, t)),
        ],
        out_specs=[
            pl.BlockSpec((1, KNN, T_SEL), lambda b, t: (b, 0, t)),
            pl.BlockSpec((F, 8), lambda b, t: (0, 0)),
        ],
        out_shape=[
            jax.ShapeDtypeStruct((B, KNN, N), jnp.int32),
            jax.ShapeDtypeStruct((F, 8), jnp.float32),
        ],
    )(x, xT, vt, ut)


def _make_sc_gather(n_idx, d):
    per_w = n_idx // NW
    nch = per_w // CH
    mesh = plsc.VectorSubcoreMesh(core_axis_name="c", subcore_axis_name="s")

    @functools.partial(
        pl.kernel,
        mesh=mesh,
        out_type=jax.ShapeDtypeStruct((n_idx, d), jnp.float32),
        scratch_types=[
            pltpu.VMEM((nch, CH), jnp.int32),
            pltpu.VMEM((CH, d), jnp.float32),
            pltpu.VMEM((CH, d), jnp.float32),
            pltpu.SemaphoreType.DMA,
        ],
        compiler_params=pltpu.CompilerParams(use_tc_tiling_on_sc=False),
    )
    def gk(table_hbm, idx_hbm, out_hbm, idx_v, buf0, buf1, sem):
        wid = jax.lax.axis_index("s") * 2 + jax.lax.axis_index("c")
        rbase = pl.multiple_of(wid * per_w, 8)
        cbase = pl.multiple_of(wid * nch, 8)
        pltpu.sync_copy(idx_hbm.at[pl.ds(cbase, nch)], idx_v)
        bufs = (buf0, buf1)
        pltpu.async_copy(table_hbm.at[idx_v.at[0]], buf0, sem)

        def outer(oc, _):
            c0 = oc * 2
            for bb in range(2):
                c = c0 + bb
                cur = bufs[bb]
                nxt = bufs[(bb + 1) % 2]

                @pl.when(c + 1 < nch)
                def _():
                    pltpu.async_copy(table_hbm.at[idx_v.at[c + 1]], nxt, sem)

                pltpu.make_async_copy(table_hbm.at[idx_v.at[c]], cur, sem).wait()
                roff = pl.multiple_of(rbase + c * CH, 8)
                pltpu.sync_copy(cur, out_hbm.at[pl.ds(roff, CH)])
            return 0

        jax.lax.fori_loop(0, nch // 2, outer, 0)

    return gk


def _gather_rows(table, idx2d):
    n_idx = idx2d.shape[0] * idx2d.shape[1]
    return _make_sc_gather(n_idx, table.shape[-1])(table, idx2d)


def _edge_body(vg_ref, u_ref, a_ref, c_ref, w2_ref, b2_ref, out_ref):
    u = u_ref[0]
    a = a_ref[...]
    c = c_ref[...]
    w2 = w2_ref[...]
    acc = jnp.full((T_EDGE, F), -1e30, jnp.float32)
    for t in range(KNN):
        vt = vg_ref[0, t]
        z = jax.nn.relu((u + vt) * a + c)
        y = jnp.dot(z, w2, preferred_element_type=jnp.float32)
        acc = jnp.maximum(acc, y)
    out_ref[0] = acc + b2_ref[...]


def _edge_call(vg4, u, a, c, w2, b2):
    nt = N // T_EDGE
    return pl.pallas_call(
        _edge_body,
        grid=(B, nt),
        in_specs=[
            pl.BlockSpec((1, KNN, T_EDGE, F), lambda b, t: (b, 0, t, 0)),
            pl.BlockSpec((1, T_EDGE, F), lambda b, t: (b, t, 0)),
            pl.BlockSpec((1, F), lambda b, t: (0, 0)),
            pl.BlockSpec((1, F), lambda b, t: (0, 0)),
            pl.BlockSpec((F, F), lambda b, t: (0, 0)),
            pl.BlockSpec((1, F), lambda b, t: (0, 0)),
        ],
        out_specs=pl.BlockSpec((1, T_EDGE, F), lambda b, t: (b, t, 0)),
        out_shape=jax.ShapeDtypeStruct((B, N, F), jnp.float32),
    )(vg4, u, a, c, w2, b2)


def _mlp_body(x_ref, w1_ref, b1_ref, w2_ref, b2_ref, w3_ref, b3_ref,
              w4_ref, b4_ref, out_ref):
    h = jax.nn.relu(jnp.dot(x_ref[...], w1_ref[...], preferred_element_type=jnp.float32) + b1_ref[...])
    h = jax.nn.relu(jnp.dot(h, w2_ref[...], preferred_element_type=jnp.float32) + b2_ref[...])
    h = jax.nn.relu(jnp.dot(h, w3_ref[...], preferred_element_type=jnp.float32) + b3_ref[...])
    o = jnp.dot(h, w4_ref[...], preferred_element_type=jnp.float32) + b4_ref[...]
    m = jnp.max(o, axis=1, keepdims=True)
    sh = o - m
    out_ref[...] = sh - jnp.log(jnp.sum(jnp.exp(sh), axis=1, keepdims=True))


def _mlp_call(x, w1, b1, w2, b2, w3, b3, w4, b4):
    rows = x.shape[0]
    nt = rows // T_MLP
    ncls = w4.shape[-1]
    return pl.pallas_call(
        _mlp_body,
        grid=(nt,),
        in_specs=[
            pl.BlockSpec((T_MLP, x.shape[1]), lambda i: (i, 0)),
            pl.BlockSpec(w1.shape, lambda i: (0, 0)),
            pl.BlockSpec((1, w1.shape[1]), lambda i: (0, 0)),
            pl.BlockSpec(w2.shape, lambda i: (0, 0)),
            pl.BlockSpec((1, w2.shape[1]), lambda i: (0, 0)),
            pl.BlockSpec(w3.shape, lambda i: (0, 0)),
            pl.BlockSpec((1, w3.shape[1]), lambda i: (0, 0)),
            pl.BlockSpec(w4.shape, lambda i: (0, 0)),
            pl.BlockSpec((1, ncls), lambda i: (0, 0)),
        ],
        out_specs=pl.BlockSpec((T_MLP, ncls), lambda i: (i, 0)),
        out_shape=jax.ShapeDtypeStruct((rows, ncls), jnp.float32),
    )(x, w1, b1, w2, b2, w3, b3, w4, b4)


def _edge_conv_layer(x, W1, b1, g, be, W2, b2):
    din = x.shape[-1]
    w1a, w1b = W1[:din], W1[din:]
    w1u = w1a - w1b
    dp = din
    if din % 8 != 0:
        pad = 8 - din % 8
        dp = din + pad
        x = jnp.pad(x, ((0, 0), (0, 0), (0, pad)))
        w1u = jnp.pad(w1u, ((0, pad), (0, 0)))
        w1b = jnp.pad(w1b, ((0, pad), (0, 0)))

    xT = jnp.swapaxes(x, 1, 2)
    u, v, ut, vt = _uv_call(x, xT, w1u, w1b, b1.reshape(1, F))

    idx, stats = _select_call(x, xT, vt, ut)

    vg = _gather_rows(v.reshape(B * N, F), idx.reshape(-1, CH))
    vg4 = vg.reshape(B, KNN, N, F)

    n_edges = B * N * KNN
    ss, sq, su, su2, sus = (stats[:, 0], stats[:, 1], stats[:, 2],
                            stats[:, 3], stats[:, 4])
    mu = (KNN * su + ss) / n_edges
    msq = (KNN * su2 + 2.0 * sus + sq) / n_edges
    var = msq - mu * mu
    a = g / jnp.sqrt(var + 1e-5)
    c = be - mu * a

    return _edge_call(vg4, u, a.reshape(1, F), c.reshape(1, F), W2,
                      b2.reshape(1, F))


def kernel(data, c1_W1, c1_b1, c1_g, c1_be, c1_W2, c1_b2, c2_W1, c2_b1, c2_g,
           c2_be, c2_W2, c2_b2, c3_W1, c3_b1, c3_g, c3_be, c3_W2, c3_b2, m_W1,
           m_b1, m_W2, m_b2, m_W3, m_b3, m_W4, m_b4):
    x0 = data
    x1 = _edge_conv_layer(x0, c1_W1, c1_b1, c1_g, c1_be, c1_W2, c1_b2)
    x2 = _edge_conv_layer(x1, c2_W1, c2_b1, c2_g, c2_be, c2_W2, c2_b2)
    x3 = _edge_conv_layer(x2, c3_W1, c3_b1, c3_g, c3_be, c3_W2, c3_b2)
    h = jnp.concatenate([x1, x2, x3], axis=-1).reshape(B * N, 3 * F)
    return _mlp_call(h, m_W1, m_b1.reshape(1, -1), m_W2, m_b2.reshape(1, -1),
                     m_W3, m_b3.reshape(1, -1), m_W4, m_b4.reshape(1, -1))


# speedup vs baseline: 1.6132x; 1.0544x over previous
import functools

import jax
import jax.numpy as jnp
from jax.experimental import pallas as pl
from jax.experimental.pallas import tpu as pltpu
from jax.experimental.pallas import tpu_sc as plsc

B = 4
N = 2048
KNN = 30
F = 64
T_SEL = 512
T_EDGE = 256
T_MLP = 512
CLEAR = 1e30
BIGCOL = 1e9
NW = 32
CH = 120


def _uv_body(x_ref, xT_ref, w1u_ref, w1v_ref, w1uT_ref, w1vT_ref, b1_ref,
             b1c_ref, u_ref, v_ref, ut_ref, vt_ref):
    x = x_ref[0]
    xT = xT_ref[0]
    u_ref[0] = jnp.dot(x, w1u_ref[...], preferred_element_type=jnp.float32) + b1_ref[...]
    v_ref[0] = jnp.dot(x, w1v_ref[...], preferred_element_type=jnp.float32)
    ut_ref[0] = jnp.dot(w1uT_ref[...], xT, preferred_element_type=jnp.float32) + b1c_ref[...]
    vt_ref[0] = jnp.dot(w1vT_ref[...], xT, preferred_element_type=jnp.float32)


def _uv_call(x, xT, w1u, w1v, b1):
    dp = x.shape[-1]
    return pl.pallas_call(
        _uv_body,
        grid=(B,),
        in_specs=[
            pl.BlockSpec((1, N, dp), lambda b: (b, 0, 0)),
            pl.BlockSpec((1, dp, N), lambda b: (b, 0, 0)),
            pl.BlockSpec((dp, F), lambda b: (0, 0)),
            pl.BlockSpec((dp, F), lambda b: (0, 0)),
            pl.BlockSpec((F, dp), lambda b: (0, 0)),
            pl.BlockSpec((F, dp), lambda b: (0, 0)),
            pl.BlockSpec((1, F), lambda b: (0, 0)),
            pl.BlockSpec((F, 1), lambda b: (0, 0)),
        ],
        out_specs=[
            pl.BlockSpec((1, N, F), lambda b: (b, 0, 0)),
            pl.BlockSpec((1, N, F), lambda b: (b, 0, 0)),
            pl.BlockSpec((1, F, N), lambda b: (b, 0, 0)),
            pl.BlockSpec((1, F, N), lambda b: (b, 0, 0)),
        ],
        out_shape=[
            jax.ShapeDtypeStruct((B, N, F), jnp.float32),
            jax.ShapeDtypeStruct((B, N, F), jnp.float32),
            jax.ShapeDtypeStruct((B, F, N), jnp.float32),
            jax.ShapeDtypeStruct((B, F, N), jnp.float32),
        ],
    )(x, xT, w1u, w1v, w1u.T, w1v.T, b1, b1.reshape(F, 1))


NCHK = 16
CW = N // NCHK
MPER = 12
RANKSTEP = 1e27
CHOSEN_MIN = 1e29
BIGP = 2e30


def _select_body(x_ref, xT_ref, vt_ref, ut_ref, idx_ref, stats_ref):
    b = pl.program_id(0)
    tile = pl.program_id(1)
    xTt = xT_ref[0]
    d2col = jnp.sum(xTt * xTt, axis=0, keepdims=True)
    boff_f = jax.lax.convert_element_type(b * N, jnp.float32)

    rowi = jax.lax.broadcasted_iota(jnp.int32, (CW, T_SEL), 0)
    coli = jax.lax.broadcasted_iota(jnp.int32, (CW, T_SEL), 1)
    rowf = rowi.astype(jnp.float32)
    subi16 = jax.lax.broadcasted_iota(jnp.int32, (16, T_SEL), 0)
    subi32 = jax.lax.broadcasted_iota(jnp.int32, (32, T_SEL), 0)

    def chunk_dist(c):
        xc = x_ref[0, c * CW:(c + 1) * CW, :]
        r = jnp.dot(xc, xTt, preferred_element_type=jnp.float32)
        d2c = jnp.sum(xc * xc, axis=1, keepdims=True)
        d = (d2c + d2col) - 2.0 * r
        return d + jnp.where(rowi + c * CW == coli + tile * T_SEL, 1e10, 0.0)

    d_enc, pvs, pis = [], [], []
    for c in range(NCHK):
        off_f = jnp.float32(c * CW) + boff_f

        def step(s, carry, off_f=off_f):
            d, pv, pi = carry
            mv = jnp.min(d, axis=0, keepdims=True)
            cand = jnp.where(d <= mv, rowf, BIGCOL)
            jr = jnp.min(cand, axis=0, keepdims=True)
            pv = jnp.where(subi16 == s, mv, pv)
            pi = jnp.where(subi16 == s, jr + off_f, pi)
            mark = CLEAR + jax.lax.convert_element_type(s, jnp.float32) * RANKSTEP
            d = jnp.where(rowf == jr, mark, d)
            return d, pv, pi

        pv0 = jnp.full((16, T_SEL), BIGP, jnp.float32)
        pi0 = jnp.full((16, T_SEL), -1.0, jnp.float32)
        d, pv, pi = jax.lax.fori_loop(0, MPER, step, (chunk_dist(c), pv0, pi0))
        d_enc.append(d)
        pvs.append(pv)
        pis.append(pi)

    P0 = jnp.concatenate(pvs, axis=0)
    PI = jnp.concatenate(pis, axis=0)

    def pstep(t, carry):
        P, pm, idxacc = carry
        wv = jnp.min(P, axis=0, keepdims=True)
        wi = jnp.min(jnp.where(P <= wv, PI, BIGCOL), axis=0, keepdims=True)
        hit = PI == wi
        pm = pm + hit.astype(jnp.float32)
        P = jnp.where(hit, BIGP, P)
        idxacc = jnp.where(subi32 == t, wi, idxacc)
        return P, pm, idxacc

    _, pm, idxacc = jax.lax.fori_loop(
        0, KNN, pstep,
        (P0, jnp.zeros((16 * NCHK, T_SEL), jnp.float32),
         jnp.zeros((32, T_SEL), jnp.float32)))

    ncs = [jnp.sum(pm[c * 16:(c + 1) * 16, :], axis=0, keepdims=True)
           for c in range(NCHK)]
    fb = jnp.max(jnp.concatenate(ncs, axis=0)) >= float(MPER)

    def normal_fn():
        parts = []
        for c in range(NCHK):
            thr = CLEAR + (ncs[c] - 0.5) * RANKSTEP
            ch = jnp.logical_and(d_enc[c] >= CHOSEN_MIN, d_enc[c] < thr)
            parts.append(ch.astype(jnp.float32))
        return (idxacc, tuple(parts))

    def fb_fn():
        def fstep(t, carry):
            idxa = carry[0]
            dd = carry[1:]
            mvs = [jnp.min(dc, axis=0, keepdims=True) for dc in dd]
            g = functools.reduce(jnp.minimum, mvs)
            cands = [jnp.min(jnp.where(dd[c] <= g, rowf + jnp.float32(c * CW),
                                       BIGCOL), axis=0, keepdims=True)
                     for c in range(NCHK)]
            wi = functools.reduce(jnp.minimum, cands)
            idxa = jnp.where(subi32 == t, wi + boff_f, idxa)
            dd = tuple(jnp.where(rowf + jnp.float32(c * CW) == wi, CLEAR, dd[c])
                       for c in range(NCHK))
            return (idxa,) + dd

        init = (jnp.zeros((32, T_SEL), jnp.float32),) + tuple(
            chunk_dist(c) for c in range(NCHK))
        out = jax.lax.fori_loop(0, KNN, fstep, init)
        parts = tuple((dc >= CHOSEN_MIN).astype(jnp.float32) for dc in out[1:])
        return (out[0], parts)

    idxf, chosen_parts = jax.lax.cond(fb, fb_fn, normal_fn)
    idx_ref[0] = idxf[:KNN].astype(jnp.int32)

    vt = vt_ref[0]
    st = jnp.zeros((F, T_SEL), jnp.float32)
    qt = jnp.zeros((F, T_SEL), jnp.float32)
    for c in range(NCHK):
        vc = vt[:, c * CW:(c + 1) * CW]
        st = st + jnp.dot(vc, chosen_parts[c], preferred_element_type=jnp.float32)
        qt = qt + jnp.dot(vc * vc, chosen_parts[c], preferred_element_type=jnp.float32)
    ut = ut_ref[0]
    c0 = jnp.sum(st, axis=1, keepdims=True)
    c1 = jnp.sum(qt, axis=1, keepdims=True)
    c2 = jnp.sum(ut, axis=1, keepdims=True)
    c3 = jnp.sum(ut * ut, axis=1, keepdims=True)
    c4 = jnp.sum(ut * st, axis=1, keepdims=True)
    stats = jnp.concatenate(
        [c0, c1, c2, c3, c4, jnp.zeros((F, 3), jnp.float32)], axis=1)

    @pl.when(jnp.logical_and(b == 0, tile == 0))
    def _():
        stats_ref[...] = jnp.zeros((F, 8), jnp.float32)

    stats_ref[...] += stats


def _select_call(x, xT, vt, ut):
    dp = x.shape[-1]
    nt = N // T_SEL
    return pl.pallas_call(
        _select_body,
        grid=(B, nt),
        in_specs=[
            pl.BlockSpec((1, N, dp), lambda b, t: (b, 0, 0)),
            pl.BlockSpec((1, dp, T_SEL), lambda b, t: (b, 0, t)),
            pl.BlockSpec((1, F, N), lambda b, t: (b, 0, 0)),
            pl.BlockSpec((1, F, T_SEL), lambda b, t: (b, 0, t)),
        ],
        out_specs=[
            pl.BlockSpec((1, KNN, T_SEL), lambda b, t: (b, 0, t)),
            pl.BlockSpec((F, 8), lambda b, t: (0, 0)),
        ],
        out_shape=[
            jax.ShapeDtypeStruct((B, KNN, N), jnp.int32),
            jax.ShapeDtypeStruct((F, 8), jnp.float32),
        ],
    )(x, xT, vt, ut)


def _make_sc_gather(n_idx, d):
    per_w = n_idx // NW
    nch = per_w // CH
    mesh = plsc.VectorSubcoreMesh(core_axis_name="c", subcore_axis_name="s")

    @functools.partial(
        pl.kernel,
        mesh=mesh,
        out_type=jax.ShapeDtypeStruct((n_idx, d), jnp.float32),
        scratch_types=[
            pltpu.VMEM((nch, CH), jnp.int32),
            pltpu.VMEM((CH, d), jnp.float32),
            pltpu.VMEM((CH, d), jnp.float32),
            pltpu.SemaphoreType.DMA,
        ],
        compiler_params=pltpu.CompilerParams(use_tc_tiling_on_sc=False),
    )
    def gk(table_hbm, idx_hbm, out_hbm, idx_v, buf0, buf1, sem):
        wid = jax.lax.axis_index("s") * 2 + jax.lax.axis_index("c")
        rbase = pl.multiple_of(wid * per_w, 8)
        cbase = pl.multiple_of(wid * nch, 8)
        pltpu.sync_copy(idx_hbm.at[pl.ds(cbase, nch)], idx_v)
        bufs = (buf0, buf1)
        pltpu.async_copy(table_hbm.at[idx_v.at[0]], buf0, sem)

        def outer(oc, _):
            c0 = oc * 2
            for bb in range(2):
                c = c0 + bb
                cur = bufs[bb]
                nxt = bufs[(bb + 1) % 2]

                @pl.when(c + 1 < nch)
                def _():
                    pltpu.async_copy(table_hbm.at[idx_v.at[c + 1]], nxt, sem)

                pltpu.make_async_copy(table_hbm.at[idx_v.at[c]], cur, sem).wait()
                roff = pl.multiple_of(rbase + c * CH, 8)
                pltpu.sync_copy(cur, out_hbm.at[pl.ds(roff, CH)])
            return 0

        jax.lax.fori_loop(0, nch // 2, outer, 0)

    return gk


def _gather_rows(table, idx2d):
    n_idx = idx2d.shape[0] * idx2d.shape[1]
    return _make_sc_gather(n_idx, table.shape[-1])(table, idx2d)


def _edge_body(vg_ref, u_ref, a_ref, c_ref, w2_ref, b2_ref, out_ref):
    u = u_ref[0]
    a = a_ref[...]
    c = c_ref[...]
    w2 = w2_ref[...]
    acc = jnp.full((T_EDGE, F), -1e30, jnp.float32)
    for t in range(KNN):
        vt = vg_ref[0, t]
        z = jax.nn.relu((u + vt) * a + c)
        y = jnp.dot(z, w2, preferred_element_type=jnp.float32)
        acc = jnp.maximum(acc, y)
    out_ref[0] = acc + b2_ref[...]


def _edge_call(vg4, u, a, c, w2, b2):
    nt = N // T_EDGE
    return pl.pallas_call(
        _edge_body,
        grid=(B, nt),
        in_specs=[
            pl.BlockSpec((1, KNN, T_EDGE, F), lambda b, t: (b, 0, t, 0)),
            pl.BlockSpec((1, T_EDGE, F), lambda b, t: (b, t, 0)),
            pl.BlockSpec((1, F), lambda b, t: (0, 0)),
            pl.BlockSpec((1, F), lambda b, t: (0, 0)),
            pl.BlockSpec((F, F), lambda b, t: (0, 0)),
            pl.BlockSpec((1, F), lambda b, t: (0, 0)),
        ],
        out_specs=pl.BlockSpec((1, T_EDGE, F), lambda b, t: (b, t, 0)),
        out_shape=jax.ShapeDtypeStruct((B, N, F), jnp.float32),
    )(vg4, u, a, c, w2, b2)


def _mlp_body(x_ref, w1_ref, b1_ref, w2_ref, b2_ref, w3_ref, b3_ref,
              w4_ref, b4_ref, out_ref):
    h = jax.nn.relu(jnp.dot(x_ref[...], w1_ref[...], preferred_element_type=jnp.float32) + b1_ref[...])
    h = jax.nn.relu(jnp.dot(h, w2_ref[...], preferred_element_type=jnp.float32) + b2_ref[...])
    h = jax.nn.relu(jnp.dot(h, w3_ref[...], preferred_element_type=jnp.float32) + b3_ref[...])
    o = jnp.dot(h, w4_ref[...], preferred_element_type=jnp.float32) + b4_ref[...]
    m = jnp.max(o, axis=1, keepdims=True)
    sh = o - m
    out_ref[...] = sh - jnp.log(jnp.sum(jnp.exp(sh), axis=1, keepdims=True))


def _mlp_call(x, w1, b1, w2, b2, w3, b3, w4, b4):
    rows = x.shape[0]
    nt = rows // T_MLP
    ncls = w4.shape[-1]
    return pl.pallas_call(
        _mlp_body,
        grid=(nt,),
        in_specs=[
            pl.BlockSpec((T_MLP, x.shape[1]), lambda i: (i, 0)),
            pl.BlockSpec(w1.shape, lambda i: (0, 0)),
            pl.BlockSpec((1, w1.shape[1]), lambda i: (0, 0)),
            pl.BlockSpec(w2.shape, lambda i: (0, 0)),
            pl.BlockSpec((1, w2.shape[1]), lambda i: (0, 0)),
            pl.BlockSpec(w3.shape, lambda i: (0, 0)),
            pl.BlockSpec((1, w3.shape[1]), lambda i: (0, 0)),
            pl.BlockSpec(w4.shape, lambda i: (0, 0)),
            pl.BlockSpec((1, ncls), lambda i: (0, 0)),
        ],
        out_specs=pl.BlockSpec((T_MLP, ncls), lambda i: (i, 0)),
        out_shape=jax.ShapeDtypeStruct((rows, ncls), jnp.float32),
    )(x, w1, b1, w2, b2, w3, b3, w4, b4)


def _edge_conv_layer(x, W1, b1, g, be, W2, b2):
    din = x.shape[-1]
    w1a, w1b = W1[:din], W1[din:]
    w1u = w1a - w1b
    dp = din
    if din % 8 != 0:
        pad = 8 - din % 8
        dp = din + pad
        x = jnp.pad(x, ((0, 0), (0, 0), (0, pad)))
        w1u = jnp.pad(w1u, ((0, pad), (0, 0)))
        w1b = jnp.pad(w1b, ((0, pad), (0, 0)))

    xT = jnp.swapaxes(x, 1, 2)
    u, v, ut, vt = _uv_call(x, xT, w1u, w1b, b1.reshape(1, F))

    idx, stats = _select_call(x, xT, vt, ut)

    vg = _gather_rows(v.reshape(B * N, F), idx.reshape(-1, CH))
    vg4 = vg.reshape(B, KNN, N, F)

    n_edges = B * N * KNN
    ss, sq, su, su2, sus = (stats[:, 0], stats[:, 1], stats[:, 2],
                            stats[:, 3], stats[:, 4])
    mu = (KNN * su + ss) / n_edges
    msq = (KNN * su2 + 2.0 * sus + sq) / n_edges
    var = msq - mu * mu
    a = g / jnp.sqrt(var + 1e-5)
    c = be - mu * a

    return _edge_call(vg4, u, a.reshape(1, F), c.reshape(1, F), W2,
                      b2.reshape(1, F))


def kernel(data, c1_W1, c1_b1, c1_g, c1_be, c1_W2, c1_b2, c2_W1, c2_b1, c2_g,
           c2_be, c2_W2, c2_b2, c3_W1, c3_b1, c3_g, c3_be, c3_W2, c3_b2, m_W1,
           m_b1, m_W2, m_b2, m_W3, m_b3, m_W4, m_b4):
    x0 = data
    x1 = _edge_conv_layer(x0, c1_W1, c1_b1, c1_g, c1_be, c1_W2, c1_b2)
    x2 = _edge_conv_layer(x1, c2_W1, c2_b1, c2_g, c2_be, c2_W2, c2_b2)
    x3 = _edge_conv_layer(x2, c3_W1, c3_b1, c3_g, c3_be, c3_W2, c3_b2)
    h = jnp.concatenate([x1, x2, x3], axis=-1).reshape(B * N, 3 * F)
    return _mlp_call(h, m_W1, m_b1.reshape(1, -1), m_W2, m_b2.reshape(1, -1),
                     m_W3, m_b3.reshape(1, -1), m_W4, m_b4.reshape(1, -1))


# speedup vs baseline: 1.6326x; 1.0120x over previous
import functools

import jax
import jax.numpy as jnp
from jax.experimental import pallas as pl
from jax.experimental.pallas import tpu as pltpu
from jax.experimental.pallas import tpu_sc as plsc

B = 4
N = 2048
KNN = 30
F = 64
T_SEL = 512
T_EDGE = 256
T_MLP = 512
CLEAR = 1e30
BIGCOL = 1e9
NW = 32
CH = 120


def _uv_body(x_ref, xT_ref, w1u_ref, w1v_ref, w1uT_ref, w1vT_ref, b1_ref,
             b1c_ref, u_ref, v_ref, ut_ref, vt_ref):
    x = x_ref[0]
    xT = xT_ref[0]
    u_ref[0] = jnp.dot(x, w1u_ref[...], preferred_element_type=jnp.float32) + b1_ref[...]
    v_ref[0] = jnp.dot(x, w1v_ref[...], preferred_element_type=jnp.float32)
    ut_ref[0] = jnp.dot(w1uT_ref[...], xT, preferred_element_type=jnp.float32) + b1c_ref[...]
    vt_ref[0] = jnp.dot(w1vT_ref[...], xT, preferred_element_type=jnp.float32)


def _uv_call(x, xT, w1u, w1v, b1):
    dp = x.shape[-1]
    return pl.pallas_call(
        _uv_body,
        grid=(B,),
        in_specs=[
            pl.BlockSpec((1, N, dp), lambda b: (b, 0, 0)),
            pl.BlockSpec((1, dp, N), lambda b: (b, 0, 0)),
            pl.BlockSpec((dp, F), lambda b: (0, 0)),
            pl.BlockSpec((dp, F), lambda b: (0, 0)),
            pl.BlockSpec((F, dp), lambda b: (0, 0)),
            pl.BlockSpec((F, dp), lambda b: (0, 0)),
            pl.BlockSpec((1, F), lambda b: (0, 0)),
            pl.BlockSpec((F, 1), lambda b: (0, 0)),
        ],
        out_specs=[
            pl.BlockSpec((1, N, F), lambda b: (b, 0, 0)),
            pl.BlockSpec((1, N, F), lambda b: (b, 0, 0)),
            pl.BlockSpec((1, F, N), lambda b: (b, 0, 0)),
            pl.BlockSpec((1, F, N), lambda b: (b, 0, 0)),
        ],
        out_shape=[
            jax.ShapeDtypeStruct((B, N, F), jnp.float32),
            jax.ShapeDtypeStruct((B, N, F), jnp.float32),
            jax.ShapeDtypeStruct((B, F, N), jnp.float32),
            jax.ShapeDtypeStruct((B, F, N), jnp.float32),
        ],
    )(x, xT, w1u, w1v, w1u.T, w1v.T, b1, b1.reshape(F, 1))


NCHK = 16
CW = N // NCHK
MPER = 12
RANKSTEP = 1e27
CHOSEN_MIN = 1e29
BIGP = 2e30


def _argmin_ax0(v, i):
    r = v.shape[0]
    while r > 1:
        h = r // 2
        lt = v[:h] <= v[h:]
        v = jnp.where(lt, v[:h], v[h:])
        i = jnp.where(lt, i[:h], i[h:])
        r = h
    return v, i


def _select_body(x_ref, xT_ref, vt_ref, ut_ref, idx_ref, stats_ref):
    b = pl.program_id(0)
    tile = pl.program_id(1)
    xTt = xT_ref[0]
    d2col = jnp.sum(xTt * xTt, axis=0, keepdims=True)
    boff_f = jax.lax.convert_element_type(b * N, jnp.float32)

    rowi = jax.lax.broadcasted_iota(jnp.int32, (CW, T_SEL), 0)
    coli = jax.lax.broadcasted_iota(jnp.int32, (CW, T_SEL), 1)
    rowf = rowi.astype(jnp.float32)
    subi16 = jax.lax.broadcasted_iota(jnp.int32, (16, T_SEL), 0)
    subi32 = jax.lax.broadcasted_iota(jnp.int32, (32, T_SEL), 0)

    def chunk_dist(c):
        xc = x_ref[0, c * CW:(c + 1) * CW, :]
        r = jnp.dot(xc, xTt, preferred_element_type=jnp.float32)
        d2c = jnp.sum(xc * xc, axis=1, keepdims=True)
        d = (d2c + d2col) - 2.0 * r
        return d + jnp.where(rowi + c * CW == coli + tile * T_SEL, 1e10, 0.0)

    d_enc, pvs, pis = [], [], []
    for c in range(NCHK):
        off_f = jnp.float32(c * CW) + boff_f

        def step(s, carry, off_f=off_f):
            d, pv, pi = carry
            mv, jr = _argmin_ax0(d, rowf)
            pv = jnp.where(subi16 == s, mv, pv)
            pi = jnp.where(subi16 == s, jr + off_f, pi)
            mark = CLEAR + jax.lax.convert_element_type(s, jnp.float32) * RANKSTEP
            d = jnp.where(rowf == jr, mark, d)
            return d, pv, pi

        pv0 = jnp.full((16, T_SEL), BIGP, jnp.float32)
        pi0 = jnp.full((16, T_SEL), -1.0, jnp.float32)
        d, pv, pi = jax.lax.fori_loop(0, MPER, step, (chunk_dist(c), pv0, pi0))
        d_enc.append(d)
        pvs.append(pv)
        pis.append(pi)

    P0 = jnp.concatenate(pvs, axis=0)
    PI = jnp.concatenate(pis, axis=0)

    def pstep(t, carry):
        P, pm, idxacc = carry
        _, wi = _argmin_ax0(P, PI)
        hit = PI == wi
        pm = pm + hit.astype(jnp.float32)
        P = jnp.where(hit, BIGP, P)
        idxacc = jnp.where(subi32 == t, wi, idxacc)
        return P, pm, idxacc

    _, pm, idxacc = jax.lax.fori_loop(
        0, KNN, pstep,
        (P0, jnp.zeros((16 * NCHK, T_SEL), jnp.float32),
         jnp.zeros((32, T_SEL), jnp.float32)))

    ncs = [jnp.sum(pm[c * 16:(c + 1) * 16, :], axis=0, keepdims=True)
           for c in range(NCHK)]
    fb = jnp.max(jnp.concatenate(ncs, axis=0)) >= float(MPER)

    def normal_fn():
        parts = []
        for c in range(NCHK):
            thr = CLEAR + (ncs[c] - 0.5) * RANKSTEP
            ch = jnp.logical_and(d_enc[c] >= CHOSEN_MIN, d_enc[c] < thr)
            parts.append(ch.astype(jnp.float32))
        return (idxacc, tuple(parts))

    def fb_fn():
        def fstep(t, carry):
            idxa = carry[0]
            dd = carry[1:]
            mvs = [jnp.min(dc, axis=0, keepdims=True) for dc in dd]
            g = functools.reduce(jnp.minimum, mvs)
            cands = [jnp.min(jnp.where(dd[c] <= g, rowf + jnp.float32(c * CW),
                                       BIGCOL), axis=0, keepdims=True)
                     for c in range(NCHK)]
            wi = functools.reduce(jnp.minimum, cands)
            idxa = jnp.where(subi32 == t, wi + boff_f, idxa)
            dd = tuple(jnp.where(rowf + jnp.float32(c * CW) == wi, CLEAR, dd[c])
                       for c in range(NCHK))
            return (idxa,) + dd

        init = (jnp.zeros((32, T_SEL), jnp.float32),) + tuple(
            chunk_dist(c) for c in range(NCHK))
        out = jax.lax.fori_loop(0, KNN, fstep, init)
        parts = tuple((dc >= CHOSEN_MIN).astype(jnp.float32) for dc in out[1:])
        return (out[0], parts)

    idxf, chosen_parts = jax.lax.cond(fb, fb_fn, normal_fn)
    idx_ref[0] = idxf[:KNN].astype(jnp.int32)

    vt = vt_ref[0]
    st = jnp.zeros((F, T_SEL), jnp.float32)
    qt = jnp.zeros((F, T_SEL), jnp.float32)
    for c in range(NCHK):
        vc = vt[:, c * CW:(c + 1) * CW]
        st = st + jnp.dot(vc, chosen_parts[c], preferred_element_type=jnp.float32)
        qt = qt + jnp.dot(vc * vc, chosen_parts[c], preferred_element_type=jnp.float32)
    ut = ut_ref[0]
    c0 = jnp.sum(st, axis=1, keepdims=True)
    c1 = jnp.sum(qt, axis=1, keepdims=True)
    c2 = jnp.sum(ut, axis=1, keepdims=True)
    c3 = jnp.sum(ut * ut, axis=1, keepdims=True)
    c4 = jnp.sum(ut * st, axis=1, keepdims=True)
    stats = jnp.concatenate(
        [c0, c1, c2, c3, c4, jnp.zeros((F, 3), jnp.float32)], axis=1)

    @pl.when(jnp.logical_and(b == 0, tile == 0))
    def _():
        stats_ref[...] = jnp.zeros((F, 8), jnp.float32)

    stats_ref[...] += stats


def _select_call(x, xT, vt, ut):
    dp = x.shape[-1]
    nt = N // T_SEL
    return pl.pallas_call(
        _select_body,
        grid=(B, nt),
        in_specs=[
            pl.BlockSpec((1, N, dp), lambda b, t: (b, 0, 0)),
            pl.BlockSpec((1, dp, T_SEL), lambda b, t: (b, 0, t)),
            pl.BlockSpec((1, F, N), lambda b, t: (b, 0, 0)),
            pl.BlockSpec((1, F, T_SEL), lambda b, t: (b, 0, t)),
        ],
        out_specs=[
            pl.BlockSpec((1, KNN, T_SEL), lambda b, t: (b, 0, t)),
            pl.BlockSpec((F, 8), lambda b, t: (0, 0)),
        ],
        out_shape=[
            jax.ShapeDtypeStruct((B, KNN, N), jnp.int32),
            jax.ShapeDtypeStruct((F, 8), jnp.float32),
        ],
    )(x, xT, vt, ut)


def _make_sc_gather(n_idx, d):
    per_w = n_idx // NW
    nch = per_w // CH
    mesh = plsc.VectorSubcoreMesh(core_axis_name="c", subcore_axis_name="s")

    @functools.partial(
        pl.kernel,
        mesh=mesh,
        out_type=jax.ShapeDtypeStruct((n_idx, d), jnp.float32),
        scratch_types=[
            pltpu.VMEM((nch, CH), jnp.int32),
            pltpu.VMEM((CH, d), jnp.float32),
            pltpu.VMEM((CH, d), jnp.float32),
            pltpu.SemaphoreType.DMA,
        ],
        compiler_params=pltpu.CompilerParams(use_tc_tiling_on_sc=False),
    )
    def gk(table_hbm, idx_hbm, out_hbm, idx_v, buf0, buf1, sem):
        wid = jax.lax.axis_index("s") * 2 + jax.lax.axis_index("c")
        rbase = pl.multiple_of(wid * per_w, 8)
        cbase = pl.multiple_of(wid * nch, 8)
        pltpu.sync_copy(idx_hbm.at[pl.ds(cbase, nch)], idx_v)
        bufs = (buf0, buf1)
        pltpu.async_copy(table_hbm.at[idx_v.at[0]], buf0, sem)

        def outer(oc, _):
            c0 = oc * 2
            for bb in range(2):
                c = c0 + bb
                cur = bufs[bb]
                nxt = bufs[(bb + 1) % 2]

                @pl.when(c + 1 < nch)
                def _():
                    pltpu.async_copy(table_hbm.at[idx_v.at[c + 1]], nxt, sem)

                pltpu.make_async_copy(table_hbm.at[idx_v.at[c]], cur, sem).wait()
                roff = pl.multiple_of(rbase + c * CH, 8)
                pltpu.sync_copy(cur, out_hbm.at[pl.ds(roff, CH)])
            return 0

        jax.lax.fori_loop(0, nch // 2, outer, 0)

    return gk


def _gather_rows(table, idx2d):
    n_idx = idx2d.shape[0] * idx2d.shape[1]
    return _make_sc_gather(n_idx, table.shape[-1])(table, idx2d)


def _edge_body(vg_ref, u_ref, a_ref, c_ref, w2_ref, b2_ref, out_ref):
    u = u_ref[0]
    a = a_ref[...]
    c = c_ref[...]
    w2 = w2_ref[...]
    acc = jnp.full((T_EDGE, F), -1e30, jnp.float32)
    for t in range(KNN):
        vt = vg_ref[0, t]
        z = jax.nn.relu((u + vt) * a + c)
        y = jnp.dot(z, w2, preferred_element_type=jnp.float32)
        acc = jnp.maximum(acc, y)
    out_ref[0] = acc + b2_ref[...]


def _edge_call(vg4, u, a, c, w2, b2):
    nt = N // T_EDGE
    return pl.pallas_call(
        _edge_body,
        grid=(B, nt),
        in_specs=[
            pl.BlockSpec((1, KNN, T_EDGE, F), lambda b, t: (b, 0, t, 0)),
            pl.BlockSpec((1, T_EDGE, F), lambda b, t: (b, t, 0)),
            pl.BlockSpec((1, F), lambda b, t: (0, 0)),
            pl.BlockSpec((1, F), lambda b, t: (0, 0)),
            pl.BlockSpec((F, F), lambda b, t: (0, 0)),
            pl.BlockSpec((1, F), lambda b, t: (0, 0)),
        ],
        out_specs=pl.BlockSpec((1, T_EDGE, F), lambda b, t: (b, t, 0)),
        out_shape=jax.ShapeDtypeStruct((B, N, F), jnp.float32),
    )(vg4, u, a, c, w2, b2)


def _mlp_body(x_ref, w1_ref, b1_ref, w2_ref, b2_ref, w3_ref, b3_ref,
              w4_ref, b4_ref, out_ref):
    h = jax.nn.relu(jnp.dot(x_ref[...], w1_ref[...], preferred_element_type=jnp.float32) + b1_ref[...])
    h = jax.nn.relu(jnp.dot(h, w2_ref[...], preferred_element_type=jnp.float32) + b2_ref[...])
    h = jax.nn.relu(jnp.dot(h, w3_ref[...], preferred_element_type=jnp.float32) + b3_ref[...])
    o = jnp.dot(h, w4_ref[...], preferred_element_type=jnp.float32) + b4_ref[...]
    m = jnp.max(o, axis=1, keepdims=True)
    sh = o - m
    out_ref[...] = sh - jnp.log(jnp.sum(jnp.exp(sh), axis=1, keepdims=True))


def _mlp_call(x, w1, b1, w2, b2, w3, b3, w4, b4):
    rows = x.shape[0]
    nt = rows // T_MLP
    ncls = w4.shape[-1]
    return pl.pallas_call(
        _mlp_body,
        grid=(nt,),
        in_specs=[
            pl.BlockSpec((T_MLP, x.shape[1]), lambda i: (i, 0)),
            pl.BlockSpec(w1.shape, lambda i: (0, 0)),
            pl.BlockSpec((1, w1.shape[1]), lambda i: (0, 0)),
            pl.BlockSpec(w2.shape, lambda i: (0, 0)),
            pl.BlockSpec((1, w2.shape[1]), lambda i: (0, 0)),
            pl.BlockSpec(w3.shape, lambda i: (0, 0)),
            pl.BlockSpec((1, w3.shape[1]), lambda i: (0, 0)),
            pl.BlockSpec(w4.shape, lambda i: (0, 0)),
            pl.BlockSpec((1, ncls), lambda i: (0, 0)),
        ],
        out_specs=pl.BlockSpec((T_MLP, ncls), lambda i: (i, 0)),
        out_shape=jax.ShapeDtypeStruct((rows, ncls), jnp.float32),
    )(x, w1, b1, w2, b2, w3, b3, w4, b4)


def _edge_conv_layer(x, W1, b1, g, be, W2, b2):
    din = x.shape[-1]
    w1a, w1b = W1[:din], W1[din:]
    w1u = w1a - w1b
    dp = din
    if din % 8 != 0:
        pad = 8 - din % 8
        dp = din + pad
        x = jnp.pad(x, ((0, 0), (0, 0), (0, pad)))
        w1u = jnp.pad(w1u, ((0, pad), (0, 0)))
        w1b = jnp.pad(w1b, ((0, pad), (0, 0)))

    xT = jnp.swapaxes(x, 1, 2)
    u, v, ut, vt = _uv_call(x, xT, w1u, w1b, b1.reshape(1, F))

    idx, stats = _select_call(x, xT, vt, ut)

    vg = _gather_rows(v.reshape(B * N, F), idx.reshape(-1, CH))
    vg4 = vg.reshape(B, KNN, N, F)

    n_edges = B * N * KNN
    ss, sq, su, su2, sus = (stats[:, 0], stats[:, 1], stats[:, 2],
                            stats[:, 3], stats[:, 4])
    mu = (KNN * su + ss) / n_edges
    msq = (KNN * su2 + 2.0 * sus + sq) / n_edges
    var = msq - mu * mu
    a = g / jnp.sqrt(var + 1e-5)
    c = be - mu * a

    return _edge_call(vg4, u, a.reshape(1, F), c.reshape(1, F), W2,
                      b2.reshape(1, F))


def kernel(data, c1_W1, c1_b1, c1_g, c1_be, c1_W2, c1_b2, c2_W1, c2_b1, c2_g,
           c2_be, c2_W2, c2_b2, c3_W1, c3_b1, c3_g, c3_be, c3_W2, c3_b2, m_W1,
           m_b1, m_W2, m_b2, m_W3, m_b3, m_W4, m_b4):
    x0 = data
    x1 = _edge_conv_layer(x0, c1_W1, c1_b1, c1_g, c1_be, c1_W2, c1_b2)
    x2 = _edge_conv_layer(x1, c2_W1, c2_b1, c2_g, c2_be, c2_W2, c2_b2)
    x3 = _edge_conv_layer(x2, c3_W1, c3_b1, c3_g, c3_be, c3_W2, c3_b2)
    h = jnp.concatenate([x1, x2, x3], axis=-1).reshape(B * N, 3 * F)
    return _mlp_call(h, m_W1, m_b1.reshape(1, -1), m_W2, m_b2.reshape(1, -1),
                     m_W3, m_b3.reshape(1, -1), m_W4, m_b4.reshape(1, -1))
